# Initial kernel scaffold; baseline (speedup 1.0000x reference)
import jax
import jax.numpy as jnp
from jax.experimental import pallas as pl


def kernel(x, edge_index, W_l, b_l, W_r):
    raise NotImplementedError("write your pallas kernel here")


# speedup vs baseline: 19.9952x; 19.9952x over previous
import functools

import jax
import jax.numpy as jnp
from jax import lax
from jax.experimental import pallas as pl
from jax.experimental.pallas import tpu as pltpu
from jax.experimental.pallas import tpu_sc as plsc

N_NODES = 100000
N_PAD = 100352
D_IN = 16
D_OUT = 32
N_EDGES = 3200000
E_PAD = 3276800
ROWS_PER_TILE_E = 800
CH = 8
N_CHUNKS = ROWS_PER_TILE_E // CH
NODE_ROWS_PER_TILE = N_PAD // 32


def _sc_aggregate_kernel():
    mesh = plsc.VectorSubcoreMesh(core_axis_name="c", subcore_axis_name="s")
    rows_per_tile = N_PAD // 16

    @functools.partial(
        pl.kernel,
        out_type=(
            jax.ShapeDtypeStruct((2, N_PAD, D_IN), jnp.float32),
            jax.ShapeDtypeStruct((2, N_PAD), jnp.float32),
        ),
        mesh=mesh,
        compiler_params=pltpu.CompilerParams(use_tc_tiling_on_sc=False),
        scratch_types=[
            pltpu.VMEM((CH, 128), jnp.int32),
            pltpu.VMEM((CH, 128), jnp.int32),
            pltpu.VMEM((CH * 128, D_IN), jnp.float32),
            pltpu.VMEM((128,), jnp.float32),
            pltpu.VMEM_SHARED((N_PAD, D_IN), jnp.float32),
            pltpu.VMEM_SHARED((N_PAD,), jnp.float32),
            pltpu.SemaphoreType.DMA,
        ],
    )
    def k(x_hbm, src_hbm, dst_hbm, z16_hbm, z1_hbm, acc_out, cnt_out,
          src_v, dst_v, rows_v, ones_v, acc_sh, cnt_sh, sem):
        c = lax.axis_index("c")
        s = lax.axis_index("s")
        tid32 = s * 2 + c

        for i in range(128 // 16):
            ones_v[pl.ds(i * 16, 16)] = jnp.ones((16,), jnp.float32)

        node0 = s * rows_per_tile
        pltpu.sync_copy(z16_hbm, acc_sh.at[pl.ds(node0, rows_per_tile)])
        pltpu.sync_copy(z1_hbm, cnt_sh.at[pl.ds(node0, rows_per_tile)])
        plsc.subcore_barrier()

        def chunk_body(g, carry):
            base = tid32 * ROWS_PER_TILE_E + g * CH
            pltpu.sync_copy(src_hbm.at[pl.ds(base, CH)], src_v)
            pltpu.sync_copy(dst_hbm.at[pl.ds(base, CH)], dst_v)
            cps = [
                pltpu.async_copy(
                    x_hbm.at[src_v.at[j]],
                    rows_v.at[pl.ds(j * 128, 128)],
                    sem,
                )
                for j in range(CH)
            ]
            for cp in cps:
                cp.wait()
            for j in range(CH):
                pltpu.sync_copy(
                    rows_v.at[pl.ds(j * 128, 128)],
                    acc_sh.at[dst_v.at[j]],
                    add=True,
                )
                pltpu.sync_copy(ones_v, cnt_sh.at[dst_v.at[j]], add=True)
            return carry

        lax.fori_loop(0, N_CHUNKS, chunk_body, 0)
        plsc.subcore_barrier()

        pltpu.sync_copy(
            acc_sh.at[pl.ds(node0, rows_per_tile)],
            acc_out.at[c].at[pl.ds(node0, rows_per_tile)],
        )
        pltpu.sync_copy(
            cnt_sh.at[pl.ds(node0, rows_per_tile)],
            cnt_out.at[c].at[pl.ds(node0, rows_per_tile)],
        )

    return k


_BLK = 8000


def _tc_epilogue(acc, cnt, x, W_l, b_l, W_r):
    n_blk = 4000
    grid = N_NODES // n_blk

    def body(acc0_ref, acc1_ref, cnt0_ref, cnt1_ref, x_ref, wl_ref, bl_ref,
             wr_ref, out_ref):
        agg = acc0_ref[...] + acc1_ref[...]
        cntv = cnt0_ref[...] + cnt1_ref[...]
        mean = agg / jnp.clip(cntv, 1.0, None)
        out = (
            jnp.dot(mean, wl_ref[...].T, preferred_element_type=jnp.float32)
            + jnp.dot(x_ref[...], wr_ref[...].T,
                      preferred_element_type=jnp.float32)
            + bl_ref[...][None, :]
        )
        out_ref[...] = out

    return pl.pallas_call(
        body,
        grid=(grid,),
        in_specs=[
            pl.BlockSpec((n_blk, D_IN), lambda i: (i, 0)),
            pl.BlockSpec((n_blk, D_IN), lambda i: (i, 0)),
            pl.BlockSpec((n_blk, 1), lambda i: (i, 0)),
            pl.BlockSpec((n_blk, 1), lambda i: (i, 0)),
            pl.BlockSpec((n_blk, D_IN), lambda i: (i, 0)),
            pl.BlockSpec((D_OUT, D_IN), lambda i: (0, 0)),
            pl.BlockSpec((D_OUT,), lambda i: (0,)),
            pl.BlockSpec((D_OUT, D_IN), lambda i: (0, 0)),
        ],
        out_specs=pl.BlockSpec((n_blk, D_OUT), lambda i: (i, 0)),
        out_shape=jax.ShapeDtypeStruct((N_NODES, D_OUT), jnp.float32),
    )(acc[0, :N_NODES], acc[1, :N_NODES],
      cnt[0, :N_NODES].reshape(N_NODES, 1), cnt[1, :N_NODES].reshape(N_NODES, 1),
      x, W_l, b_l, W_r)


def kernel(x, edge_index, W_l, b_l, W_r):
    ei = edge_index.astype(jnp.int32)
    pad_e = E_PAD - N_EDGES
    src = jnp.concatenate([ei[0], jnp.zeros((pad_e,), jnp.int32)])
    dst = jnp.concatenate(
        [ei[1], jnp.full((pad_e,), N_NODES, jnp.int32)])
    src2d = src.reshape(E_PAD // 128, 128)
    dst2d = dst.reshape(E_PAD // 128, 128)
    z16 = jnp.zeros((N_PAD // 16, D_IN), jnp.float32)
    z1 = jnp.zeros((N_PAD // 16,), jnp.float32)

    acc, cnt = _sc_aggregate_kernel()(x, src2d, dst2d, z16, z1)
    return _tc_epilogue(acc, cnt, x, W_l, b_l, W_r)
